# baseline (speedup 1.0000x reference)
import functools

import jax
import jax.numpy as jnp
from jax import lax
from jax.experimental import pallas as pl
from jax.experimental.pallas import tpu as pltpu
from jax.experimental.pallas import tpu_sc as plsc

CROP_H = 384
CROP_W = 384
CHUNK = 64
PAD_H = CHUNK + 8
NBUF = 2
NUM_TILES = 32
NVREG = CROP_W // 16


def _scalar_at(vec_ref, i):
    splat_idx = jnp.full((16,), i, dtype=jnp.int32)
    return jnp.max(plsc.load_gather(vec_ref, [splat_idx]))


def _crop_body(C, imgs_per_tile, batch_hbm, top_hbm, left_hbm, out_hbm,
               top_v, left_v, vin0, vin1, vout0, vout1, sem_in, sem_out):
    H, W = batch_hbm.shape[2], batch_hbm.shape[3]
    vin = [vin0, vin1]
    vout = [vout0, vout1]
    wid = lax.axis_index("s") * 2 + lax.axis_index("c")
    pltpu.sync_copy(top_hbm, top_v)
    pltpu.sync_copy(left_hbm, left_v)
    lanes = lax.iota(jnp.int32, 16)

    chunks_per_img = CROP_H // CHUNK
    nu = imgs_per_tile * chunks_per_img

    def scalars(u):
        j = u // chunks_per_img
        k = u % chunks_per_img
        img = wid * imgs_per_tile + j
        b = img // C
        c = img % C
        t = _scalar_at(top_v, b)
        l = _scalar_at(left_v, b)
        t8 = jnp.minimum((t // 8) * 8, H - CROP_H - 8)
        dt = t - t8
        col_vec = jnp.full((16,), l, dtype=jnp.int32) + lanes
        return b, c, t8 + k * CHUNK, dt, col_vec, k * CHUNK

    def start_in(s, p):
        b, c, row_src = s[0], s[1], s[2]
        pltpu.make_async_copy(
            batch_hbm.at[b, c, pl.ds(row_src, PAD_H), :],
            vin[p],
            sem_in.at[p],
        ).start()

    def start_out(s, p):
        b, c, row_dst = s[0], s[1], s[5]
        pltpu.make_async_copy(
            vout[p],
            out_hbm.at[b, c, pl.ds(row_dst, CHUNK), :],
            sem_out.at[p],
        ).start()

    def wait_in(p):
        pltpu.make_async_copy(
            batch_hbm.at[0, 0, pl.ds(0, PAD_H), :],
            vin[p],
            sem_in.at[p],
        ).wait()

    def wait_out(p):
        pltpu.make_async_copy(
            vout[p],
            out_hbm.at[0, 0, pl.ds(0, CHUNK), :],
            sem_out.at[p],
        ).wait()

    def shift(s, p):
        dt, col_vec = s[3], s[4]
        src = vin[p]
        dst = vout[p]

        @plsc.parallel_loop(0, CHUNK, step=1, unroll=1)
        def row_body(r):
            row_idx = jnp.full((16,), r + dt, dtype=jnp.int32)
            for jv in range(NVREG):
                x = plsc.load_gather(src, [row_idx, col_vec + (16 * jv)])
                dst[r, pl.ds(16 * jv, 16)] = x

    start_in(scalars(0), 0)
    start_in(scalars(1), 1)
    for u in range(NBUF):
        wait_in(u % NBUF)
        s = scalars(u)
        shift(s, u % NBUF)
        start_in(scalars(u + NBUF), u % NBUF)
        start_out(s, u % NBUF)

    def steady(i, carry):
        for p in range(NBUF):
            u = i * NBUF + p
            wait_in(p)
            wait_out(p)
            s = scalars(u)
            shift(s, p)
            start_in(scalars(u + NBUF), p)
            start_out(s, p)
        return carry

    lax.fori_loop(1, (nu // NBUF) - 1, steady, 0)

    for u in range(nu - NBUF, nu):
        p = u % NBUF
        wait_in(p)
        wait_out(p)
        s = scalars(u)
        shift(s, p)
        start_out(s, p)
    for u in range(nu - NBUF, nu):
        wait_out(u % NBUF)


def kernel(batch, top, left):
    B, C, H, W = batch.shape
    imgs_per_tile = (B * C) // NUM_TILES
    mesh = plsc.VectorSubcoreMesh(core_axis_name="c", subcore_axis_name="s")
    f = pl.kernel(
        functools.partial(_crop_body, C, imgs_per_tile),
        out_type=jax.ShapeDtypeStruct((B, C, CROP_H, CROP_W), batch.dtype),
        mesh=mesh,
        scratch_types=[
            pltpu.VMEM((B,), jnp.int32),
            pltpu.VMEM((B,), jnp.int32),
            pltpu.VMEM((PAD_H, W), jnp.float32),
            pltpu.VMEM((PAD_H, W), jnp.float32),
            pltpu.VMEM((CHUNK, CROP_W), jnp.float32),
            pltpu.VMEM((CHUNK, CROP_W), jnp.float32),
            pltpu.SemaphoreType.DMA((NBUF,)),
            pltpu.SemaphoreType.DMA((NBUF,)),
        ],
        compiler_params=pltpu.CompilerParams(needs_layout_passes=False),
    )
    return f(batch, top, left)


# speedup vs baseline: 1.0033x; 1.0033x over previous
import functools

import jax
import jax.numpy as jnp
from jax import lax
from jax.experimental import pallas as pl
from jax.experimental.pallas import tpu as pltpu
from jax.experimental.pallas import tpu_sc as plsc

CROP_H = 384
CROP_W = 384
CHUNK = 64
PAD_H = CHUNK + 8
NBUF = 2
NUM_TILES = 32
NVREG = CROP_W // 16


def _scalar_at(vec_ref, i):
    splat_idx = jnp.full((16,), i, dtype=jnp.int32)
    return jnp.max(plsc.load_gather(vec_ref, [splat_idx]))


def _crop_body(C, imgs_per_tile, batch_hbm, top_hbm, left_hbm, out_hbm,
               top_v, left_v, vin0, vin1, vout0, vout1, sem_in, sem_out):
    H, W = batch_hbm.shape[2], batch_hbm.shape[3]
    vin = [vin0, vin1]
    vout = [vout0, vout1]
    wid = lax.axis_index("s") * 2 + lax.axis_index("c")
    pltpu.sync_copy(top_hbm, top_v)
    pltpu.sync_copy(left_hbm, left_v)
    lanes = lax.iota(jnp.int32, 16)

    chunks_per_img = CROP_H // CHUNK
    nu = imgs_per_tile * chunks_per_img

    def _bck(u):
        j = u // chunks_per_img
        k = u % chunks_per_img
        img = wid * imgs_per_tile + j
        return img // C, img % C, k

    def scalars(u):
        b, c, k = _bck(u)
        splat_b = jnp.full((16,), b, dtype=jnp.int32)
        t_vec = plsc.load_gather(top_v, [splat_b])
        l_vec = plsc.load_gather(left_v, [splat_b])
        t8_vec = jnp.minimum((t_vec // 8) * 8, H - CROP_H - 8)
        dt_vec = t_vec - t8_vec
        col_vec = l_vec + lanes
        return b, c, k, dt_vec, col_vec

    def in_row_src(u):
        b, c, k = _bck(u)
        t = _scalar_at(top_v, b)
        t8 = jnp.minimum((t // 8) * 8, H - CROP_H - 8)
        return b, c, t8 + k * CHUNK

    def start_in(u, p):
        b, c, row_src = in_row_src(u)
        pltpu.make_async_copy(
            batch_hbm.at[b, c, pl.ds(row_src, PAD_H), :],
            vin[p],
            sem_in.at[p],
        ).start()

    def start_out(s, p):
        b, c, row_dst = s[0], s[1], s[2] * CHUNK
        pltpu.make_async_copy(
            vout[p],
            out_hbm.at[b, c, pl.ds(row_dst, CHUNK), :],
            sem_out.at[p],
        ).start()

    def wait_in(p):
        pltpu.make_async_copy(
            batch_hbm.at[0, 0, pl.ds(0, PAD_H), :],
            vin[p],
            sem_in.at[p],
        ).wait()

    def wait_out(p):
        pltpu.make_async_copy(
            vout[p],
            out_hbm.at[0, 0, pl.ds(0, CHUNK), :],
            sem_out.at[p],
        ).wait()

    def shift(s, p):
        dt_vec, col_vec = s[3], s[4]
        src = vin[p]
        dst = vout[p]

        @plsc.parallel_loop(0, CHUNK, step=1, unroll=1)
        def row_body(r):
            row_idx = dt_vec + r
            for jv in range(NVREG):
                x = plsc.load_gather(src, [row_idx, col_vec + (16 * jv)])
                dst[r, pl.ds(16 * jv, 16)] = x

    start_in(0, 0)
    start_in(1, 1)
    for u in range(NBUF):
        wait_in(u % NBUF)
        s = scalars(u)
        shift(s, u % NBUF)
        start_in(u + NBUF, u % NBUF)
        start_out(s, u % NBUF)

    def steady(i, carry):
        for p in range(NBUF):
            u = i * NBUF + p
            wait_in(p)
            wait_out(p)
            s = scalars(u)
            shift(s, p)
            start_in(u + NBUF, p)
            start_out(s, p)
        return carry

    lax.fori_loop(1, (nu // NBUF) - 1, steady, 0)

    for u in range(nu - NBUF, nu):
        p = u % NBUF
        wait_in(p)
        wait_out(p)
        s = scalars(u)
        shift(s, p)
        start_out(s, p)
    for u in range(nu - NBUF, nu):
        wait_out(u % NBUF)


def kernel(batch, top, left):
    B, C, H, W = batch.shape
    imgs_per_tile = (B * C) // NUM_TILES
    mesh = plsc.VectorSubcoreMesh(core_axis_name="c", subcore_axis_name="s")
    f = pl.kernel(
        functools.partial(_crop_body, C, imgs_per_tile),
        out_type=jax.ShapeDtypeStruct((B, C, CROP_H, CROP_W), batch.dtype),
        mesh=mesh,
        scratch_types=[
            pltpu.VMEM((B,), jnp.int32),
            pltpu.VMEM((B,), jnp.int32),
            pltpu.VMEM((PAD_H, W), jnp.float32),
            pltpu.VMEM((PAD_H, W), jnp.float32),
            pltpu.VMEM((CHUNK, CROP_W), jnp.float32),
            pltpu.VMEM((CHUNK, CROP_W), jnp.float32),
            pltpu.SemaphoreType.DMA((NBUF,)),
            pltpu.SemaphoreType.DMA((NBUF,)),
        ],
        compiler_params=pltpu.CompilerParams(needs_layout_passes=False),
    )
    return f(batch, top, left)
